# baseline (speedup 1.0000x reference)
import functools

import jax
import jax.numpy as jnp
from jax import lax
from jax.experimental import pallas as pl
from jax.experimental.pallas import tpu as pltpu
from jax.experimental.pallas import tpu_sc as plsc

E = 16
S = 2048
DI = 768
DH = 512
DO = 768
TILE = 256
NT = S // TILE
GRID = NT + E - 1


def _cumsum(a, axis):
    n = a.shape[axis]
    sh = 1
    while sh < n:
        zshape = list(a.shape)
        zshape[axis] = sh
        shifted = jnp.concatenate(
            [jnp.zeros(zshape, a.dtype), lax.slice_in_dim(a, 0, n - sh, axis=axis)],
            axis=axis)
        a = a + shifted
        sh *= 2
    return a


def _router_body(x_ref, wr_ref, br_ref, probs_ref, pos_ref, se_ref,
                 tile_ref, exp_ref, valid_ref):
    x = x_ref[...]
    wr = wr_ref[...]
    logits = lax.dot_general(x, wr, (((1,), (1,)), ((), ())),
                             preferred_element_type=jnp.float32)
    logits = logits + br_ref[...]

    m = jnp.max(logits, axis=1, keepdims=True)
    p = jnp.exp(logits - m)
    probs_ref[...] = p / jnp.sum(p, axis=1, keepdims=True)

    ecol = lax.broadcasted_iota(jnp.int32, (S, E), 1)
    e_t = jnp.min(jnp.where(logits == m, ecol, E), axis=1, keepdims=True)
    oh = (ecol == e_t).astype(jnp.int32)
    csum = _cumsum(oh, 0)
    counts = csum[S - 1:S, :]
    end = _cumsum(counts, 1)
    off = end - counts

    pos_ref[...] = jnp.sum(oh * (off + csum - 1), axis=1, keepdims=True)

    prow = lax.broadcasted_iota(jnp.int32, (S, E), 0)
    se_ref[...] = jnp.sum((prow >= end).astype(jnp.int32), axis=1,
                          keepdims=True)

    nonempty = counts > 0
    t0 = off // TILE
    t1 = jnp.where(nonempty, (off + counts - 1) // TILE, 0)
    t0 = jnp.where(nonempty, t0, 0)
    npairs = jnp.where(nonempty, t1 - t0 + 1, 0)
    pend = _cumsum(npairs, 1)
    poff = pend - npairs
    srow = lax.broadcasted_iota(jnp.int32, (GRID, E), 0)
    hit = (srow >= poff) & (srow < pend)
    ecol2 = lax.broadcasted_iota(jnp.int32, (GRID, E), 1)
    exp_s = jnp.sum(jnp.where(hit, ecol2, 0), axis=1, keepdims=True)
    tile_s = jnp.sum(jnp.where(hit, t0 + (srow - poff), 0), axis=1,
                     keepdims=True)
    valid_s = jnp.sum(hit.astype(jnp.int32), axis=1, keepdims=True)

    last_e = jnp.sum((end <= S - 1).astype(jnp.int32), axis=1, keepdims=True)
    valid_b = valid_s > 0
    tile_ref[...] = jnp.where(valid_b, tile_s, NT - 1)
    exp_ref[...] = jnp.where(valid_b, exp_s, last_e)
    valid_ref[...] = valid_s


_router_call = pl.pallas_call(
    _router_body,
    out_shape=[
        jax.ShapeDtypeStruct((S, E), jnp.float32),
        jax.ShapeDtypeStruct((S, 1), jnp.int32),
        jax.ShapeDtypeStruct((S, 1), jnp.int32),
        jax.ShapeDtypeStruct((GRID, 1), jnp.int32),
        jax.ShapeDtypeStruct((GRID, 1), jnp.int32),
        jax.ShapeDtypeStruct((GRID, 1), jnp.int32),
    ],
)


_NC, _NS = 2, 16
_NW = _NC * _NS
_RPW = S // _NW

_NCH = 4
_RPC = _RPW // _NCH

@functools.cache
def _sc_kernels():
    mesh = plsc.VectorSubcoreMesh(core_axis_name="c", subcore_axis_name="s",
                                  num_cores=_NC, num_subcores=_NS)

    @functools.partial(
        pl.kernel,
        out_type=jax.ShapeDtypeStruct((S, DI), jnp.float32),
        mesh=mesh,
        scratch_types=[
            pltpu.VMEM((_NCH, _RPC), jnp.int32),
            pltpu.VMEM((_RPW, DI), jnp.float32),
            pltpu.SemaphoreType.DMA,
            pltpu.SemaphoreType.DMA,
            pltpu.SemaphoreType.DMA,
            pltpu.SemaphoreType.DMA,
            pltpu.SemaphoreType.DMA,
        ],
    )
    def _sc_scatter(x_hbm, pos3_hbm, out_hbm, idx_v, rows_v,
                    sl0, sl1, sl2, sl3, ss):
        wid = lax.axis_index("s") * _NC + lax.axis_index("c")
        base = wid * _RPW
        pltpu.sync_copy(pos3_hbm.at[wid], idx_v)
        lsems = [sl0, sl1, sl2, sl3]
        loads = [
            pltpu.async_copy(x_hbm.at[pl.ds(base + c * _RPC, _RPC)],
                             rows_v.at[pl.ds(c * _RPC, _RPC)], lsems[c])
            for c in range(_NCH)
        ]
        stores = []
        for c in range(_NCH):
            loads[c].wait()
            stores.append(
                pltpu.async_copy(rows_v.at[pl.ds(c * _RPC, _RPC)],
                                 out_hbm.at[idx_v.at[c]], ss))
        for st in stores:
            st.wait()

    @functools.partial(
        pl.kernel,
        out_type=jax.ShapeDtypeStruct((S, DO), jnp.float32),
        mesh=mesh,
        scratch_types=[
            pltpu.VMEM((_NCH, _RPC), jnp.int32),
            pltpu.VMEM((_RPW, DO), jnp.float32),
            pltpu.SemaphoreType.DMA,
            pltpu.SemaphoreType.DMA,
            pltpu.SemaphoreType.DMA,
            pltpu.SemaphoreType.DMA,
            pltpu.SemaphoreType.DMA,
        ],
    )
    def _sc_gather(y_hbm, pos3_hbm, out_hbm, idx_v, rows_v,
                   sg0, sg1, sg2, sg3, ss):
        wid = lax.axis_index("s") * _NC + lax.axis_index("c")
        base = wid * _RPW
        pltpu.sync_copy(pos3_hbm.at[wid], idx_v)
        gsems = [sg0, sg1, sg2, sg3]
        gets = [
            pltpu.async_copy(y_hbm.at[idx_v.at[c]],
                             rows_v.at[pl.ds(c * _RPC, _RPC)], gsems[c])
            for c in range(_NCH)
        ]
        puts = []
        for c in range(_NCH):
            gets[c].wait()
            puts.append(
                pltpu.async_copy(rows_v.at[pl.ds(c * _RPC, _RPC)],
                                 out_hbm.at[pl.ds(base + c * _RPC, _RPC)], ss))
        for p in puts:
            p.wait()

    return _sc_scatter, _sc_gather


def _moe_body(tile_t, exp_t, valid_t, x_ref, w1_ref, b1_ref, w2_ref, b2_ref,
              se_ref, out_ref):
    s = pl.program_id(0)
    prev = tile_t[jnp.maximum(s - 1, 0)]

    @pl.when(jnp.logical_or(s == 0, tile_t[s] != prev))
    def _init():
        out_ref[...] = jnp.zeros_like(out_ref)

    @pl.when(valid_t[s] > 0)
    def _compute():
        x = x_ref[...]
        h = lax.dot_general(x, w1_ref[0], (((1,), (1,)), ((), ())),
                            preferred_element_type=jnp.float32)
        h = h + b1_ref[0]
        h = 0.5 * h * (1.0 + lax.erf(h * 0.7071067811865476))
        y = lax.dot_general(h, w2_ref[0], (((1,), (1,)), ((), ())),
                            preferred_element_type=jnp.float32)
        y = y + b2_ref[0]
        mask = (se_ref[0] == exp_t[s]).astype(jnp.float32)
        out_ref[...] += y * mask


_moe_grid = pltpu.PrefetchScalarGridSpec(
    num_scalar_prefetch=3,
    grid=(GRID,),
    in_specs=[
        pl.BlockSpec((TILE, DI), lambda s, tt, et, vt: (tt[s], 0)),
        pl.BlockSpec((1, DH, DI), lambda s, tt, et, vt: (et[s], 0, 0)),
        pl.BlockSpec((1, 1, DH), lambda s, tt, et, vt: (et[s], 0, 0)),
        pl.BlockSpec((1, DO, DH), lambda s, tt, et, vt: (et[s], 0, 0)),
        pl.BlockSpec((1, 1, DO), lambda s, tt, et, vt: (et[s], 0, 0)),
        pl.BlockSpec((1, TILE, 1), lambda s, tt, et, vt: (tt[s], 0, 0)),
    ],
    out_specs=pl.BlockSpec((TILE, DO), lambda s, tt, et, vt: (tt[s], 0)),
)

_moe_call = pl.pallas_call(
    _moe_body,
    grid_spec=_moe_grid,
    out_shape=jax.ShapeDtypeStruct((S, DO), jnp.float32),
)


def kernel(x, Wr, br, W1, b1, W2, b2):
    x2 = x.reshape(S, DI)
    probs, pos2, se2, tile_t, exp_t, valid_t = _router_call(
        x2, Wr, br.reshape(1, E))
    pos3 = pos2.reshape(_NW, _NCH, _RPC)
    _sc_scatter, _sc_gather = _sc_kernels()
    sorted_x = _sc_scatter(x2, pos3)
    y_sorted = _moe_call(
        tile_t.reshape(GRID), exp_t.reshape(GRID), valid_t.reshape(GRID),
        sorted_x, W1, b1.reshape(E, 1, DH), W2, b2.reshape(E, 1, DO),
        se2.reshape(NT, TILE, 1))
    out = _sc_gather(y_sorted, pos3)
    return out.reshape(1, S, DO), probs.reshape(1, S, E)


# speedup vs baseline: 1.0026x; 1.0026x over previous
import functools

import jax
import jax.numpy as jnp
from jax import lax
from jax.experimental import pallas as pl
from jax.experimental.pallas import tpu as pltpu
from jax.experimental.pallas import tpu_sc as plsc

E = 16
S = 2048
DI = 768
DH = 512
DO = 768
TILE = 256
NT = S // TILE
GRID = NT + E - 1


def _cumsum(a, axis):
    n = a.shape[axis]
    sh = 1
    while sh < n:
        zshape = list(a.shape)
        zshape[axis] = sh
        shifted = jnp.concatenate(
            [jnp.zeros(zshape, a.dtype), lax.slice_in_dim(a, 0, n - sh, axis=axis)],
            axis=axis)
        a = a + shifted
        sh *= 2
    return a


def _router_body(x_ref, wr_ref, br_ref, probs_ref, pos_ref, se_ref,
                 tile_ref, exp_ref, valid_ref):
    x = x_ref[...]
    wr = wr_ref[...]
    logits = lax.dot_general(x, wr, (((1,), (1,)), ((), ())),
                             preferred_element_type=jnp.float32)
    logits = logits + br_ref[...]

    m = jnp.max(logits, axis=1, keepdims=True)
    p = jnp.exp(logits - m)
    probs_ref[...] = p / jnp.sum(p, axis=1, keepdims=True)

    ecol = lax.broadcasted_iota(jnp.int32, (S, E), 1)
    e_t = jnp.min(jnp.where(logits == m, ecol, E), axis=1, keepdims=True)
    oh = (ecol == e_t).astype(jnp.int32)
    csum = _cumsum(oh, 0)
    counts = csum[S - 1:S, :]
    end = _cumsum(counts, 1)
    off = end - counts

    pos_ref[...] = jnp.sum(oh * (off + csum - 1), axis=1, keepdims=True)

    prow = lax.broadcasted_iota(jnp.int32, (S, E), 0)
    se_ref[...] = jnp.sum((prow >= end).astype(jnp.int32), axis=1,
                          keepdims=True)

    nonempty = counts > 0
    t0 = off // TILE
    t1 = jnp.where(nonempty, (off + counts - 1) // TILE, 0)
    t0 = jnp.where(nonempty, t0, 0)
    npairs = jnp.where(nonempty, t1 - t0 + 1, 0)
    pend = _cumsum(npairs, 1)
    poff = pend - npairs
    srow = lax.broadcasted_iota(jnp.int32, (GRID, E), 0)
    hit = (srow >= poff) & (srow < pend)
    ecol2 = lax.broadcasted_iota(jnp.int32, (GRID, E), 1)
    exp_s = jnp.sum(jnp.where(hit, ecol2, 0), axis=1, keepdims=True)
    tile_s = jnp.sum(jnp.where(hit, t0 + (srow - poff), 0), axis=1,
                     keepdims=True)
    valid_s = jnp.sum(hit.astype(jnp.int32), axis=1, keepdims=True)

    last_e = jnp.sum((end <= S - 1).astype(jnp.int32), axis=1, keepdims=True)
    valid_b = valid_s > 0
    tile_ref[...] = jnp.where(valid_b, tile_s, NT - 1)
    exp_ref[...] = jnp.where(valid_b, exp_s, last_e)
    valid_ref[...] = valid_s


_router_call = pl.pallas_call(
    _router_body,
    out_shape=[
        jax.ShapeDtypeStruct((S, E), jnp.float32),
        jax.ShapeDtypeStruct((S, 1), jnp.int32),
        jax.ShapeDtypeStruct((S, 1), jnp.int32),
        jax.ShapeDtypeStruct((GRID, 1), jnp.int32),
        jax.ShapeDtypeStruct((GRID, 1), jnp.int32),
        jax.ShapeDtypeStruct((GRID, 1), jnp.int32),
    ],
)


_NC, _NS = 2, 16
_NW = _NC * _NS
_RPW = S // _NW

@functools.cache
def _sc_kernels():
    mesh = plsc.VectorSubcoreMesh(core_axis_name="c", subcore_axis_name="s",
                                  num_cores=_NC, num_subcores=_NS)

    @functools.partial(
        pl.kernel,
        out_type=jax.ShapeDtypeStruct((S, DI), jnp.float32),
        mesh=mesh,
        scratch_types=[
            pltpu.VMEM((_RPW,), jnp.int32),
            pltpu.VMEM((_RPW, DI), jnp.float32),
            pltpu.SemaphoreType.DMA,
        ],
    )
    def _sc_scatter(x_hbm, pos_hbm, out_hbm, idx_v, rows_v, sem):
        wid = lax.axis_index("s") * _NC + lax.axis_index("c")
        base = wid * _RPW
        pltpu.sync_copy(pos_hbm.at[pl.ds(base, _RPW)], idx_v)
        pltpu.sync_copy(x_hbm.at[pl.ds(base, _RPW)], rows_v)
        pltpu.async_copy(rows_v, out_hbm.at[idx_v], sem).wait()

    @functools.partial(
        pl.kernel,
        out_type=jax.ShapeDtypeStruct((S, DO), jnp.float32),
        mesh=mesh,
        scratch_types=[
            pltpu.VMEM((_RPW,), jnp.int32),
            pltpu.VMEM((_RPW, DO), jnp.float32),
            pltpu.SemaphoreType.DMA,
        ],
    )
    def _sc_gather(y_hbm, pos_hbm, out_hbm, idx_v, rows_v, sem):
        wid = lax.axis_index("s") * _NC + lax.axis_index("c")
        base = wid * _RPW
        pltpu.sync_copy(pos_hbm.at[pl.ds(base, _RPW)], idx_v)
        pltpu.async_copy(y_hbm.at[idx_v], rows_v, sem).wait()
        pltpu.sync_copy(rows_v, out_hbm.at[pl.ds(base, _RPW)])

    return _sc_scatter, _sc_gather


def _moe_body(tile_t, exp_t, valid_t, x_ref, w1_ref, b1_ref, w2_ref, b2_ref,
              se_ref, out_ref):
    s = pl.program_id(0)
    prev = tile_t[jnp.maximum(s - 1, 0)]

    @pl.when(jnp.logical_or(s == 0, tile_t[s] != prev))
    def _init():
        out_ref[...] = jnp.zeros_like(out_ref)

    @pl.when(valid_t[s] > 0)
    def _compute():
        x = x_ref[...]
        h = lax.dot_general(x, w1_ref[0], (((1,), (1,)), ((), ())),
                            preferred_element_type=jnp.float32)
        h = h + b1_ref[0]
        h = 0.5 * h * (1.0 + lax.erf(h * 0.7071067811865476))
        y = lax.dot_general(h, w2_ref[0], (((1,), (1,)), ((), ())),
                            preferred_element_type=jnp.float32)
        y = y + b2_ref[0]
        mask = (se_ref[0] == exp_t[s]).astype(jnp.float32)
        out_ref[...] += y * mask


_moe_grid = pltpu.PrefetchScalarGridSpec(
    num_scalar_prefetch=3,
    grid=(GRID,),
    in_specs=[
        pl.BlockSpec((TILE, DI), lambda s, tt, et, vt: (tt[s], 0)),
        pl.BlockSpec((1, DH, DI), lambda s, tt, et, vt: (et[s], 0, 0)),
        pl.BlockSpec((1, 1, DH), lambda s, tt, et, vt: (et[s], 0, 0)),
        pl.BlockSpec((1, DO, DH), lambda s, tt, et, vt: (et[s], 0, 0)),
        pl.BlockSpec((1, 1, DO), lambda s, tt, et, vt: (et[s], 0, 0)),
        pl.BlockSpec((1, TILE, 1), lambda s, tt, et, vt: (tt[s], 0, 0)),
    ],
    out_specs=pl.BlockSpec((TILE, DO), lambda s, tt, et, vt: (tt[s], 0)),
)

_moe_call = pl.pallas_call(
    _moe_body,
    grid_spec=_moe_grid,
    out_shape=jax.ShapeDtypeStruct((S, DO), jnp.float32),
)


def kernel(x, Wr, br, W1, b1, W2, b2):
    x2 = x.reshape(S, DI)
    probs, pos2, se2, tile_t, exp_t, valid_t = _router_call(
        x2, Wr, br.reshape(1, E))
    pos = pos2.reshape(S)
    _sc_scatter, _sc_gather = _sc_kernels()
    sorted_x = _sc_scatter(x2, pos)
    y_sorted = _moe_call(
        tile_t.reshape(GRID), exp_t.reshape(GRID), valid_t.reshape(GRID),
        sorted_x, W1, b1.reshape(E, 1, DH), W2, b2.reshape(E, 1, DO),
        se2.reshape(NT, TILE, 1))
    out = _sc_gather(y_sorted, pos)
    return out.reshape(1, S, DO), probs.reshape(1, S, E)
